# baseline (speedup 1.0000x reference)
import functools

import jax
import jax.numpy as jnp
from jax import lax
from jax.experimental import pallas as pl
from jax.experimental.pallas import tpu as pltpu
from jax.experimental.pallas import tpu_sc as plsc

N = 10000
E = 320000
D = 128
L = 8
NC = 2
NS = 16
NW = NC * NS
EPT = E // NW
CH = 80
NCH = EPT // CH
TL = EPT - NCH * CH
DCH = 80


def _sc_mesh():
    return plsc.VectorSubcoreMesh(
        core_axis_name="c", subcore_axis_name="s",
        num_cores=NC, num_subcores=NS)


def _deg_body(dst_hbm, out_hbm, *scr):
    c = lax.axis_index("c")
    s = lax.axis_index("s")
    wid = s * NC + c
    didx = scr[0:6]
    ones_v = scr[6]
    tmp_v = scr[7]
    acc_sh = scr[8]
    semd = scr[9:15]
    sems = scr[15:18]

    for k in range(DCH // 16):
        ones_v[pl.ds(k * 16, 16)] = jnp.ones((16,), jnp.float32)

    Z = 1000

    @pl.when(s < 10)
    def _zero():
        for k in range(Z // 16):
            tmp_v[pl.ds(k * 16, 16)] = jnp.zeros((16,), jnp.float32)
        tmp_v[pl.ds(Z - 16, 16)] = jnp.zeros((16,), jnp.float32)
        pltpu.sync_copy(tmp_v, acc_sh.at[pl.ds(pl.multiple_of(s * Z, 8), Z)])

    plsc.subcore_barrier()

    DN = EPT // DCH

    def load(j):
        base = pl.multiple_of(wid * EPT + j * DCH, 8)
        return pltpu.async_copy(dst_hbm.at[pl.ds(base, DCH)],
                                didx[j % 6], semd[j % 6])

    hi = [None] * 6
    hs = [None] * 3
    for j in range(min(4, DN)):
        hi[j % 6] = load(j)
    for j in range(DN):
        hi[j % 6].wait()
        b3 = j % 3
        if hs[b3] is not None:
            hs[b3].wait()
        hs[b3] = pltpu.async_copy(ones_v, acc_sh.at[didx[j % 6]],
                                  sems[b3], add=True)
        if j + 4 < DN:
            ob = (j - 2) % 3
            if j >= 2 and hs[ob] is not None:
                hs[ob].wait()
                hs[ob] = None
            hi[(j + 4) % 6] = load(j + 4)
    for b in range(3):
        if hs[b] is not None:
            hs[b].wait()

    plsc.subcore_barrier()

    @pl.when(s < 10)
    def _out():
        src0 = pl.multiple_of(s * Z, 8)
        dst0 = pl.multiple_of(c * N + s * Z, 8)
        pltpu.sync_copy(acc_sh.at[pl.ds(src0, Z)], tmp_v)
        pltpu.sync_copy(tmp_v, out_hbm.at[pl.ds(dst0, Z)])


def _deg_call(dst):
    f = pl.kernel(
        _deg_body,
        out_type=jax.ShapeDtypeStruct((NC * N,), jnp.float32),
        mesh=_sc_mesh(),
        scratch_types=(
            [pltpu.VMEM((DCH,), jnp.int32)] * 6
            + [pltpu.VMEM((DCH,), jnp.float32)]
            + [pltpu.VMEM((1000,), jnp.float32)]
            + [pltpu.VMEM_SHARED((N,), jnp.float32)]
            + [pltpu.SemaphoreType.DMA] * 9
        ),
    )
    return f(dst)


NBI = 8
NBR = 4


def _agg_body(h_hbm, src_hbm, dst_hbm, out_hbm, *scr):
    c = lax.axis_index("c")
    s = lax.axis_index("s")
    wid = s * NC + c
    sidx = scr[0:NBI]
    didx = scr[NBI:2 * NBI]
    rows = scr[2 * NBI:2 * NBI + NBR]
    acc_sh = scr[2 * NBI + NBR]
    semi = scr[2 * NBI + NBR + 1:2 * NBI + NBR + 1 + NBI]
    semg = scr[2 * NBI + NBR + 1 + NBI:2 * NBI + NBR + 1 + NBI + NBR]
    sems = scr[2 * NBI + NBR + 1 + NBI + NBR:]
    rows0 = rows[0]

    RZ = 40

    @pl.when(s < 10)
    def _zero():
        for q in range(D // 16):
            for k in range(RZ):
                rows0[k, pl.ds(q * 16, 16)] = jnp.zeros((16,), jnp.float32)
        hz = []
        for k in range(1000 // RZ):
            r0 = pl.multiple_of(s * 1000 + k * RZ, 8)
            hz.append(pltpu.async_copy(rows0.at[pl.ds(0, RZ)],
                                       acc_sh.at[pl.ds(r0, RZ)], semg[0]))
        for h in hz:
            h.wait()

    plsc.subcore_barrier()

    def load_idx(j):
        b = j % NBI
        base = pl.multiple_of(wid * EPT + j * CH, 8)
        h1 = pltpu.async_copy(src_hbm.at[pl.ds(base, CH)], sidx[b], semi[b])
        h2 = pltpu.async_copy(dst_hbm.at[pl.ds(base, CH)], didx[b], semi[b])
        return (h1, h2)

    def start_gather(j):
        b = j % NBR
        return pltpu.async_copy(h_hbm.at[sidx[j % NBI]], rows[b], semg[b])

    def start_scatter(j):
        b = j % NBR
        return pltpu.async_copy(rows[b], acc_sh.at[didx[j % NBI]],
                                sems[b], add=True)

    PFI = NBI - 2
    PFG = NBR - 1
    hi = [None] * NBI
    hg = [None] * NBR
    hs = [None] * NBR
    for j in range(min(PFI, NCH)):
        hi[j % NBI] = load_idx(j)
    for j in range(min(PFG, NCH)):
        for h in hi[j % NBI]:
            h.wait()
        hg[j % NBR] = start_gather(j)

    for j in range(NCH):
        b = j % NBR
        hg[b].wait()
        hs[b] = start_scatter(j)
        if j + PFI < NCH:
            ob = (j - 2) % NBR
            if j >= 2 and hs[ob] is not None:
                hs[ob].wait()
                hs[ob] = None
            hi[(j + PFI) % NBI] = load_idx(j + PFI)
        if j + PFG < NCH:
            gb = (j + PFG) % NBR
            if hs[gb] is not None:
                hs[gb].wait()
                hs[gb] = None
            for h in hi[(j + PFG) % NBI]:
                h.wait()
            hg[gb] = start_gather(j + PFG)
    for b in range(NBR):
        if hs[b] is not None:
            hs[b].wait()

    plsc.subcore_barrier()

    @pl.when(s < 10)
    def _out():
        r0 = pl.multiple_of(s * 1000, 8)
        d0 = pl.multiple_of(c * N + s * 1000, 8)
        pltpu.sync_copy(acc_sh.at[pl.ds(r0, 1000)], out_hbm.at[pl.ds(d0, 1000)])


def _agg_call(h, src, dst):
    f = pl.kernel(
        _agg_body,
        out_type=jax.ShapeDtypeStruct((NC * N, D), jnp.float32),
        mesh=_sc_mesh(),
        scratch_types=(
            [pltpu.VMEM((CH,), jnp.int32)] * (2 * NBI)
            + [pltpu.VMEM((CH, D), jnp.float32)] * NBR
            + [pltpu.VMEM_SHARED((N, D), jnp.float32)]
            + [pltpu.SemaphoreType.DMA] * (NBI + 2 * NBR)
        ),
    )
    return f(h, src, dst)


BN = 10000
NG = N // BN


def _dense_body(h_ref, pa_ref, pb_ref, inv_ref, ws_ref, wn_ref, b_ref, o_ref):
    agg = (pa_ref[...] + pb_ref[...]) * inv_ref[...]
    acc = jnp.dot(h_ref[...], ws_ref[...], preferred_element_type=jnp.float32)
    acc = acc + jnp.dot(agg, wn_ref[...], preferred_element_type=jnp.float32)
    o_ref[...] = jnp.maximum(acc + b_ref[...], 0.0)


def _dense_call(h, parts, inv, ws, wn, b):
    return pl.pallas_call(
        _dense_body,
        grid=(NG,),
        in_specs=[
            pl.BlockSpec((BN, D), lambda i: (i, 0)),
            pl.BlockSpec((BN, D), lambda i: (i, 0)),
            pl.BlockSpec((BN, D), lambda i: (i + NG, 0)),
            pl.BlockSpec((BN, 1), lambda i: (i, 0)),
            pl.BlockSpec((D, D), lambda i: (0, 0)),
            pl.BlockSpec((D, D), lambda i: (0, 0)),
            pl.BlockSpec((1, D), lambda i: (0, 0)),
        ],
        out_specs=pl.BlockSpec((BN, D), lambda i: (i, 0)),
        out_shape=jax.ShapeDtypeStruct((N, D), jnp.float32),
    )(h, parts, parts, inv, ws, wn, b)


def kernel(x, edge_index, W_neigh, W_self, bias):
    src = edge_index[0]
    dst = edge_index[1]
    deg2 = _deg_call(dst)
    inv = (1.0 / jnp.maximum(deg2[:N] + deg2[N:], 1.0))[:, None]
    h = x
    for i in range(L):
        parts = _agg_call(h, src, dst)
        h = _dense_call(h, parts, inv, W_self[i], W_neigh[i],
                        bias[i].reshape(1, D))
    return h


# speedup vs baseline: 1.0015x; 1.0015x over previous
import functools

import jax
import jax.numpy as jnp
from jax import lax
from jax.experimental import pallas as pl
from jax.experimental.pallas import tpu as pltpu
from jax.experimental.pallas import tpu_sc as plsc

N = 10000
E = 320000
D = 128
L = 8
NC = 2
NS = 16
NW = NC * NS
EPT = E // NW
CH = 80
NCH = EPT // CH
TL = EPT - NCH * CH
DCH = 80


def _sc_mesh():
    return plsc.VectorSubcoreMesh(
        core_axis_name="c", subcore_axis_name="s",
        num_cores=NC, num_subcores=NS)


def _deg_body(dst_hbm, out_hbm, *scr):
    c = lax.axis_index("c")
    s = lax.axis_index("s")
    wid = s * NC + c
    didx = scr[0:6]
    ones_v = scr[6]
    tmp_v = scr[7]
    acc_sh = scr[8]
    semd = scr[9:15]
    sems = scr[15:18]

    for k in range(DCH // 16):
        ones_v[pl.ds(k * 16, 16)] = jnp.ones((16,), jnp.float32)

    Z = 1000

    @pl.when(s < 10)
    def _zero():
        for k in range(Z // 16):
            tmp_v[pl.ds(k * 16, 16)] = jnp.zeros((16,), jnp.float32)
        tmp_v[pl.ds(Z - 16, 16)] = jnp.zeros((16,), jnp.float32)
        pltpu.sync_copy(tmp_v, acc_sh.at[pl.ds(pl.multiple_of(s * Z, 8), Z)])

    plsc.subcore_barrier()

    DN = EPT // DCH

    def load(j):
        base = pl.multiple_of(wid * EPT + j * DCH, 8)
        return pltpu.async_copy(dst_hbm.at[pl.ds(base, DCH)],
                                didx[j % 6], semd[j % 6])

    hi = [None] * 6
    hs = [None] * 3
    for j in range(min(4, DN)):
        hi[j % 6] = load(j)
    for j in range(DN):
        hi[j % 6].wait()
        b3 = j % 3
        if hs[b3] is not None:
            hs[b3].wait()
        hs[b3] = pltpu.async_copy(ones_v, acc_sh.at[didx[j % 6]],
                                  sems[b3], add=True)
        if j + 4 < DN:
            ob = (j - 2) % 3
            if j >= 2 and hs[ob] is not None:
                hs[ob].wait()
                hs[ob] = None
            hi[(j + 4) % 6] = load(j + 4)
    for b in range(3):
        if hs[b] is not None:
            hs[b].wait()

    plsc.subcore_barrier()

    @pl.when(s < 10)
    def _out():
        src0 = pl.multiple_of(s * Z, 8)
        dst0 = pl.multiple_of(c * N + s * Z, 8)
        pltpu.sync_copy(acc_sh.at[pl.ds(src0, Z)], tmp_v)
        pltpu.sync_copy(tmp_v, out_hbm.at[pl.ds(dst0, Z)])


def _deg_call(dst):
    f = pl.kernel(
        _deg_body,
        out_type=jax.ShapeDtypeStruct((NC * N,), jnp.float32),
        mesh=_sc_mesh(),
        scratch_types=(
            [pltpu.VMEM((DCH,), jnp.int32)] * 6
            + [pltpu.VMEM((DCH,), jnp.float32)]
            + [pltpu.VMEM((1000,), jnp.float32)]
            + [pltpu.VMEM_SHARED((N,), jnp.float32)]
            + [pltpu.SemaphoreType.DMA] * 9
        ),
    )
    return f(dst)


NBI = 8
NBR = 4


def _agg_body(h_hbm, src_hbm, dst_hbm, out_hbm, *scr):
    c = lax.axis_index("c")
    s = lax.axis_index("s")
    wid = s * NC + c
    sidx = scr[0:NBI]
    didx = scr[NBI:2 * NBI]
    rows = scr[2 * NBI:2 * NBI + NBR]
    acc_sh = scr[2 * NBI + NBR]
    semi = scr[2 * NBI + NBR + 1:2 * NBI + NBR + 1 + NBI]
    semg = scr[2 * NBI + NBR + 1 + NBI:2 * NBI + NBR + 1 + NBI + NBR]
    sems = scr[2 * NBI + NBR + 1 + NBI + NBR:]
    rows0 = rows[0]

    RZ = 40

    @pl.when(s < 10)
    def _zero():
        for q in range(D // 16):
            for k in range(RZ):
                rows0[k, pl.ds(q * 16, 16)] = jnp.zeros((16,), jnp.float32)
        hz = []
        for k in range(1000 // RZ):
            r0 = pl.multiple_of(s * 1000 + k * RZ, 8)
            hz.append(pltpu.async_copy(rows0.at[pl.ds(0, RZ)],
                                       acc_sh.at[pl.ds(r0, RZ)], semg[0]))
        for h in hz:
            h.wait()

    plsc.subcore_barrier()

    def load_idx(j):
        b = j % NBI
        base = pl.multiple_of(wid * EPT + j * CH, 8)
        h1 = pltpu.async_copy(src_hbm.at[pl.ds(base, CH)], sidx[b], semi[b])
        h2 = pltpu.async_copy(dst_hbm.at[pl.ds(base, CH)], didx[b], semi[b])
        return (h1, h2)

    def start_gather(j):
        b = j % NBR
        return pltpu.async_copy(h_hbm.at[sidx[j % NBI]], rows[b], semg[b])

    def start_scatter(j):
        b = j % NBR
        return pltpu.async_copy(rows[b], acc_sh.at[didx[j % NBI]],
                                sems[b], add=True)

    PFI = NBI - 2
    PFG = NBR - 1
    hi = [None] * NBI
    hg = [None] * NBR
    hs = [None] * NBR
    for j in range(min(PFI, NCH)):
        hi[j % NBI] = load_idx(j)
    for j in range(min(PFG, NCH)):
        for h in hi[j % NBI]:
            h.wait()
        hg[j % NBR] = start_gather(j)

    for j in range(NCH):
        b = j % NBR
        hg[b].wait()
        hs[b] = start_scatter(j)
        if j + PFI < NCH:
            ob = (j - 2) % NBR
            if j >= 2 and hs[ob] is not None:
                hs[ob].wait()
                hs[ob] = None
            hi[(j + PFI) % NBI] = load_idx(j + PFI)
        if j + PFG < NCH:
            gb = (j + PFG) % NBR
            if hs[gb] is not None:
                hs[gb].wait()
                hs[gb] = None
            for h in hi[(j + PFG) % NBI]:
                h.wait()
            hg[gb] = start_gather(j + PFG)
    for b in range(NBR):
        if hs[b] is not None:
            hs[b].wait()

    plsc.subcore_barrier()

    @pl.when(s < 10)
    def _out():
        r0 = pl.multiple_of(s * 1000, 8)
        d0 = pl.multiple_of(c * N + s * 1000, 8)
        pltpu.sync_copy(acc_sh.at[pl.ds(r0, 1000)], out_hbm.at[pl.ds(d0, 1000)])


def _agg_call(h, src, dst):
    f = pl.kernel(
        _agg_body,
        out_type=jax.ShapeDtypeStruct((NC * N, D), jnp.float32),
        mesh=_sc_mesh(),
        scratch_types=(
            [pltpu.VMEM((CH,), jnp.int32)] * (2 * NBI)
            + [pltpu.VMEM((CH, D), jnp.float32)] * NBR
            + [pltpu.VMEM_SHARED((N, D), jnp.float32)]
            + [pltpu.SemaphoreType.DMA] * (NBI + 2 * NBR)
        ),
    )
    return f(h, src, dst)


BN = 2000
NG = N // BN


def _dense_body(h_ref, pa_ref, pb_ref, inv_ref, ws_ref, wn_ref, b_ref, o_ref):
    agg = (pa_ref[...] + pb_ref[...]) * inv_ref[...]
    acc = jnp.dot(h_ref[...], ws_ref[...], preferred_element_type=jnp.float32)
    acc = acc + jnp.dot(agg, wn_ref[...], preferred_element_type=jnp.float32)
    o_ref[...] = jnp.maximum(acc + b_ref[...], 0.0)


def _dense_call(h, parts, inv, ws, wn, b):
    return pl.pallas_call(
        _dense_body,
        grid=(NG,),
        in_specs=[
            pl.BlockSpec((BN, D), lambda i: (i, 0)),
            pl.BlockSpec((BN, D), lambda i: (i, 0)),
            pl.BlockSpec((BN, D), lambda i: (i + NG, 0)),
            pl.BlockSpec((BN, 1), lambda i: (i, 0)),
            pl.BlockSpec((D, D), lambda i: (0, 0)),
            pl.BlockSpec((D, D), lambda i: (0, 0)),
            pl.BlockSpec((1, D), lambda i: (0, 0)),
        ],
        out_specs=pl.BlockSpec((BN, D), lambda i: (i, 0)),
        out_shape=jax.ShapeDtypeStruct((N, D), jnp.float32),
    )(h, parts, parts, inv, ws, wn, b)


def kernel(x, edge_index, W_neigh, W_self, bias):
    src = edge_index[0]
    dst = edge_index[1]
    deg2 = _deg_call(dst)
    inv = (1.0 / jnp.maximum(deg2[:N] + deg2[N:], 1.0))[:, None]
    h = x
    for i in range(L):
        parts = _agg_call(h, src, dst)
        h = _dense_call(h, parts, inv, W_self[i], W_neigh[i],
                        bias[i].reshape(1, D))
    return h
